# Initial kernel scaffold; baseline (speedup 1.0000x reference)
import jax
import jax.numpy as jnp
from jax import lax
from jax.experimental import pallas as pl
from jax.experimental.pallas import tpu as pltpu
from jax.experimental.pallas import tpu_sc as plsc

N = 10000
E = 320000
D = 128
G = 128
DOUT = 16

NC = 2
NS = 16
NW = NC * NS
N_PAD = 10240
ROWS_PER_SUB = N_PAD // NS
EB = 128
E_PAD = 327680
EPT = E_PAD // NW
NBLK = EPT // EB


def _deg_body(dst_hbm, zd_hbm, ones_hbm, out_hbm, didx_v, ones_v, acc):
    c = lax.axis_index("c")
    s = lax.axis_index("s")
    wid = c * NS + s
    pltpu.sync_copy(ones_hbm, ones_v)
    pltpu.sync_copy(zd_hbm, acc.at[pl.ds(s * ROWS_PER_SUB, ROWS_PER_SUB)])
    plsc.subcore_barrier()

    @pl.loop(0, NBLK)
    def _(b):
        off = wid * EPT + b * EB
        pltpu.sync_copy(dst_hbm.at[pl.ds(off, EB)], didx_v)
        pltpu.sync_copy(ones_v, acc.at[didx_v], add=True)

    plsc.subcore_barrier()
    sl = pl.ds(s * ROWS_PER_SUB, ROWS_PER_SUB)
    pltpu.sync_copy(acc.at[sl], out_hbm.at[c, sl])


def _agg_body(u_hbm, src_hbm, dst_hbm, zc_hbm, out_hbm,
              sidx_v, didx_v, rows_v, acc):
    c = lax.axis_index("c")
    s = lax.axis_index("s")
    wid = c * NS + s
    pltpu.sync_copy(zc_hbm, acc.at[pl.ds(s * ROWS_PER_SUB, ROWS_PER_SUB)])
    plsc.subcore_barrier()

    @pl.loop(0, NBLK)
    def _(b):
        off = wid * EPT + b * EB
        pltpu.sync_copy(src_hbm.at[pl.ds(off, EB)], sidx_v)
        pltpu.sync_copy(dst_hbm.at[pl.ds(off, EB)], didx_v)
        pltpu.sync_copy(u_hbm.at[sidx_v], rows_v)
        pltpu.sync_copy(rows_v, acc.at[didx_v], add=True)

    plsc.subcore_barrier()
    sl = pl.ds(s * ROWS_PER_SUB, ROWS_PER_SUB)
    pltpu.sync_copy(acc.at[sl], out_hbm.at[c, sl])


_SC_MESH = plsc.VectorSubcoreMesh(core_axis_name="c", subcore_axis_name="s")

_deg_sc = pl.kernel(
    _deg_body,
    out_type=jax.ShapeDtypeStruct((NC, N_PAD, 16), jnp.float32),
    mesh=_SC_MESH,
    scratch_types=[
        pltpu.VMEM((EB,), jnp.int32),
        pltpu.VMEM((EB, 16), jnp.float32),
        pltpu.VMEM_SHARED((N_PAD, 16), jnp.float32),
    ],
)

_agg_sc = pl.kernel(
    _agg_body,
    out_type=jax.ShapeDtypeStruct((NC, N_PAD, D), jnp.float32),
    mesh=_SC_MESH,
    scratch_types=[
        pltpu.VMEM((EB,), jnp.int32),
        pltpu.VMEM((EB,), jnp.int32),
        pltpu.VMEM((EB, D), jnp.float32),
        pltpu.VMEM_SHARED((N_PAD, D), jnp.float32),
    ],
)


RA = 2048


def _lin1_kern(x_ref, w_ref, deg_ref, u_ref, dinv_ref):
    deg = deg_ref[0, :, 0] + deg_ref[1, :, 0] + 1.0
    dinv = lax.rsqrt(deg)
    h = jnp.dot(x_ref[...], w_ref[...], preferred_element_type=jnp.float32)
    u_ref[...] = h * dinv[:, None]
    dinv_ref[...] = dinv[:, None]


def _layer2_kern(agg_ref, u_ref, dinv_ref, b1_ref, w_ref, u2_ref):
    a = agg_ref[0] + agg_ref[1] + u_ref[...]
    h = jnp.maximum(a * dinv_ref[...] + b1_ref[...], 0.0)
    u2_ref[...] = jnp.dot(h, w_ref[...],
                          preferred_element_type=jnp.float32) * dinv_ref[...]


def _final_kern(agg_ref, u_ref, dinv_ref, b2_ref, batch_ref, wfc_ref, bfc_ref,
                out_ref, pool_ref, cnt_ref):
    i = pl.program_id(0)

    @pl.when(i == 0)
    def _():
        pool_ref[...] = jnp.zeros_like(pool_ref)
        cnt_ref[...] = jnp.zeros_like(cnt_ref)

    a = agg_ref[0] + agg_ref[1] + u_ref[...]
    h = jnp.maximum(a * dinv_ref[...] + b2_ref[...], 0.0)
    b = batch_ref[0]
    rows = lax.broadcasted_iota(jnp.int32, (G, RA), 0)
    oh = (rows == b).astype(jnp.float32)
    pool_ref[...] += jnp.dot(oh, h, preferred_element_type=jnp.float32)
    cnt_ref[...] += jnp.dot(oh, jnp.ones((RA, 1), jnp.float32),
                            preferred_element_type=jnp.float32)

    @pl.when(i == pl.num_programs(0) - 1)
    def _():
        g = pool_ref[...] / jnp.maximum(cnt_ref[...], 1.0)
        logits = jnp.dot(g, wfc_ref[...],
                         preferred_element_type=jnp.float32) + bfc_ref[...]
        m = jnp.max(logits, axis=1, keepdims=True)
        e = jnp.exp(logits - m)
        out_ref[...] = (logits - m) - jnp.log(jnp.sum(e, axis=1, keepdims=True))


_NB = N_PAD // RA

_lin1 = pl.pallas_call(
    _lin1_kern,
    grid=(_NB,),
    in_specs=[
        pl.BlockSpec((RA, D), lambda i: (i, 0)),
        pl.BlockSpec((D, D), lambda i: (0, 0)),
        pl.BlockSpec((NC, RA, 16), lambda i: (0, i, 0)),
    ],
    out_specs=[
        pl.BlockSpec((RA, D), lambda i: (i, 0)),
        pl.BlockSpec((RA, 1), lambda i: (i, 0)),
    ],
    out_shape=[
        jax.ShapeDtypeStruct((N_PAD, D), jnp.float32),
        jax.ShapeDtypeStruct((N_PAD, 1), jnp.float32),
    ],
)

_layer2 = pl.pallas_call(
    _layer2_kern,
    grid=(_NB,),
    in_specs=[
        pl.BlockSpec((NC, RA, D), lambda i: (0, i, 0)),
        pl.BlockSpec((RA, D), lambda i: (i, 0)),
        pl.BlockSpec((RA, 1), lambda i: (i, 0)),
        pl.BlockSpec((1, D), lambda i: (0, 0)),
        pl.BlockSpec((D, D), lambda i: (0, 0)),
    ],
    out_specs=pl.BlockSpec((RA, D), lambda i: (i, 0)),
    out_shape=jax.ShapeDtypeStruct((N_PAD, D), jnp.float32),
)

_final = pl.pallas_call(
    _final_kern,
    grid=(_NB,),
    in_specs=[
        pl.BlockSpec((NC, RA, D), lambda i: (0, i, 0)),
        pl.BlockSpec((RA, D), lambda i: (i, 0)),
        pl.BlockSpec((RA, 1), lambda i: (i, 0)),
        pl.BlockSpec((1, D), lambda i: (0, 0)),
        pl.BlockSpec((1, 1, RA), lambda i: (i, 0, 0)),
        pl.BlockSpec((D, DOUT), lambda i: (0, 0)),
        pl.BlockSpec((1, DOUT), lambda i: (0, 0)),
    ],
    out_specs=pl.BlockSpec((G, DOUT), lambda i: (0, 0)),
    out_shape=jax.ShapeDtypeStruct((G, DOUT), jnp.float32),
    scratch_shapes=[
        pltpu.VMEM((G, G), jnp.float32),
        pltpu.VMEM((G, 1), jnp.float32),
    ],
)


def kernel(x, edge_index, batch, W1, b1, W2, b2, Wfc, bfc):
    src = edge_index[0].astype(jnp.int32)
    dst = edge_index[1].astype(jnp.int32)
    padk = jnp.full((E_PAD - E,), N, dtype=jnp.int32)
    src_pad = jnp.concatenate([src, padk])
    dst_pad = jnp.concatenate([dst, padk])
    x_pad = jnp.pad(x, ((0, N_PAD - N), (0, 0)))
    batch_pad = jnp.concatenate(
        [batch.astype(jnp.int32), jnp.full((N_PAD - N,), G, dtype=jnp.int32)]
    ).reshape(_NB, 1, RA)
    zc = jnp.zeros((ROWS_PER_SUB, D), jnp.float32)
    zd = jnp.zeros((ROWS_PER_SUB, 16), jnp.float32)
    ones16 = jnp.ones((EB, 16), jnp.float32)

    deg16 = _deg_sc(dst_pad, zd, ones16)
    u1, dinv = _lin1(x_pad, W1, deg16)
    agg1 = _agg_sc(u1, src_pad, dst_pad, zc)
    u2 = _layer2(agg1, u1, dinv, b1.reshape(1, D), W2)
    agg2 = _agg_sc(u2, src_pad, dst_pad, zc)
    out = _final(agg2, u2, dinv, b2.reshape(1, D), batch_pad,
                 Wfc, bfc.reshape(1, DOUT))
    return out


# speedup vs baseline: 6.9202x; 6.9202x over previous
import jax
import jax.numpy as jnp
from jax import lax
from jax.experimental import pallas as pl
from jax.experimental.pallas import tpu as pltpu
from jax.experimental.pallas import tpu_sc as plsc

N = 10000
E = 320000
D = 128
G = 128
DOUT = 16

NC = 2
NS = 16
NW = NC * NS
N_PAD = 10240
ROWS_PER_SUB = N_PAD // NS
EB = 128
E_PAD = 327680
EPT = E_PAD // NW
NBLK = EPT // EB


def _deg_body(dst_hbm, zc_hbm, ones_hbm, out_hbm, didx_v, ones_v, acc):
    c = lax.axis_index("c")
    s = lax.axis_index("s")
    wid = c * NS + s
    pltpu.sync_copy(ones_hbm, ones_v)
    pltpu.sync_copy(zc_hbm, acc.at[pl.ds(s * ROWS_PER_SUB, ROWS_PER_SUB)])
    plsc.subcore_barrier()

    @pl.loop(0, NBLK)
    def _(b):
        off = wid * EPT + b * EB
        pltpu.sync_copy(dst_hbm.at[pl.ds(off, EB)], didx_v)
        pltpu.sync_copy(ones_v, acc.at[didx_v], add=True)

    plsc.subcore_barrier()
    sl = pl.ds(s * ROWS_PER_SUB, ROWS_PER_SUB)
    pltpu.sync_copy(acc.at[sl], out_hbm.at[c, sl])


def _agg_body(u_hbm, src_hbm, dst_hbm, zc_hbm, out_hbm,
              sidx_v, didx_v, rows_v, acc):
    c = lax.axis_index("c")
    s = lax.axis_index("s")
    wid = c * NS + s
    pltpu.sync_copy(zc_hbm, acc.at[pl.ds(s * ROWS_PER_SUB, ROWS_PER_SUB)])
    plsc.subcore_barrier()

    @pl.loop(0, NBLK)
    def _(b):
        off = wid * EPT + b * EB
        pltpu.sync_copy(src_hbm.at[pl.ds(off, EB)], sidx_v)
        pltpu.sync_copy(dst_hbm.at[pl.ds(off, EB)], didx_v)
        pltpu.sync_copy(u_hbm.at[sidx_v], rows_v)
        pltpu.sync_copy(rows_v, acc.at[didx_v], add=True)

    plsc.subcore_barrier()
    sl = pl.ds(s * ROWS_PER_SUB, ROWS_PER_SUB)
    pltpu.sync_copy(acc.at[sl], out_hbm.at[c, sl])


_SC_MESH = plsc.VectorSubcoreMesh(core_axis_name="c", subcore_axis_name="s")

_deg_sc = pl.kernel(
    _deg_body,
    out_type=jax.ShapeDtypeStruct((NC, N_PAD, D), jnp.float32),
    mesh=_SC_MESH,
    scratch_types=[
        pltpu.VMEM((EB,), jnp.int32),
        pltpu.VMEM((EB, D), jnp.float32),
        pltpu.VMEM_SHARED((N_PAD, D), jnp.float32),
    ],
)

_agg_sc = pl.kernel(
    _agg_body,
    out_type=jax.ShapeDtypeStruct((NC, N_PAD, D), jnp.float32),
    mesh=_SC_MESH,
    scratch_types=[
        pltpu.VMEM((EB,), jnp.int32),
        pltpu.VMEM((EB,), jnp.int32),
        pltpu.VMEM((EB, D), jnp.float32),
        pltpu.VMEM_SHARED((N_PAD, D), jnp.float32),
    ],
)


RA = 2048


def _lin1_kern(x_ref, w_ref, deg_ref, u_ref, dinv_ref):
    deg = deg_ref[0, :, 0] + deg_ref[1, :, 0] + 1.0
    dinv = lax.rsqrt(deg)
    h = jnp.dot(x_ref[...], w_ref[...], preferred_element_type=jnp.float32)
    u_ref[...] = h * dinv[:, None]
    dinv_ref[...] = dinv[:, None]


def _layer2_kern(agg_ref, u_ref, dinv_ref, b1_ref, w_ref, u2_ref):
    a = agg_ref[0] + agg_ref[1] + u_ref[...]
    h = jnp.maximum(a * dinv_ref[...] + b1_ref[...], 0.0)
    u2_ref[...] = jnp.dot(h, w_ref[...],
                          preferred_element_type=jnp.float32) * dinv_ref[...]


def _final_kern(agg_ref, u_ref, dinv_ref, b2_ref, batch_ref, wfc_ref, bfc_ref,
                out_ref, pool_ref, cnt_ref):
    i = pl.program_id(0)

    @pl.when(i == 0)
    def _():
        pool_ref[...] = jnp.zeros_like(pool_ref)
        cnt_ref[...] = jnp.zeros_like(cnt_ref)

    a = agg_ref[0] + agg_ref[1] + u_ref[...]
    h = jnp.maximum(a * dinv_ref[...] + b2_ref[...], 0.0)
    b = batch_ref[0]
    rows = lax.broadcasted_iota(jnp.int32, (G, RA), 0)
    oh = (rows == b).astype(jnp.float32)
    pool_ref[...] += jnp.dot(oh, h, preferred_element_type=jnp.float32)
    cnt_ref[...] += jnp.dot(oh, jnp.ones((RA, 1), jnp.float32),
                            preferred_element_type=jnp.float32)

    @pl.when(i == pl.num_programs(0) - 1)
    def _():
        g = pool_ref[...] / jnp.maximum(cnt_ref[...], 1.0)
        logits = jnp.dot(g, wfc_ref[...],
                         preferred_element_type=jnp.float32) + bfc_ref[...]
        m = jnp.max(logits, axis=1, keepdims=True)
        e = jnp.exp(logits - m)
        out_ref[...] = (logits - m) - jnp.log(jnp.sum(e, axis=1, keepdims=True))


_NB = N_PAD // RA

_lin1 = pl.pallas_call(
    _lin1_kern,
    grid=(_NB,),
    in_specs=[
        pl.BlockSpec((RA, D), lambda i: (i, 0)),
        pl.BlockSpec((D, D), lambda i: (0, 0)),
        pl.BlockSpec((NC, RA, D), lambda i: (0, i, 0)),
    ],
    out_specs=[
        pl.BlockSpec((RA, D), lambda i: (i, 0)),
        pl.BlockSpec((RA, 1), lambda i: (i, 0)),
    ],
    out_shape=[
        jax.ShapeDtypeStruct((N_PAD, D), jnp.float32),
        jax.ShapeDtypeStruct((N_PAD, 1), jnp.float32),
    ],
)

_layer2 = pl.pallas_call(
    _layer2_kern,
    grid=(_NB,),
    in_specs=[
        pl.BlockSpec((NC, RA, D), lambda i: (0, i, 0)),
        pl.BlockSpec((RA, D), lambda i: (i, 0)),
        pl.BlockSpec((RA, 1), lambda i: (i, 0)),
        pl.BlockSpec((1, D), lambda i: (0, 0)),
        pl.BlockSpec((D, D), lambda i: (0, 0)),
    ],
    out_specs=pl.BlockSpec((RA, D), lambda i: (i, 0)),
    out_shape=jax.ShapeDtypeStruct((N_PAD, D), jnp.float32),
)

_final = pl.pallas_call(
    _final_kern,
    grid=(_NB,),
    in_specs=[
        pl.BlockSpec((NC, RA, D), lambda i: (0, i, 0)),
        pl.BlockSpec((RA, D), lambda i: (i, 0)),
        pl.BlockSpec((RA, 1), lambda i: (i, 0)),
        pl.BlockSpec((1, D), lambda i: (0, 0)),
        pl.BlockSpec((1, 1, RA), lambda i: (i, 0, 0)),
        pl.BlockSpec((D, DOUT), lambda i: (0, 0)),
        pl.BlockSpec((1, DOUT), lambda i: (0, 0)),
    ],
    out_specs=pl.BlockSpec((G, DOUT), lambda i: (0, 0)),
    out_shape=jax.ShapeDtypeStruct((G, DOUT), jnp.float32),
    scratch_shapes=[
        pltpu.VMEM((G, G), jnp.float32),
        pltpu.VMEM((G, 1), jnp.float32),
    ],
)


def kernel(x, edge_index, batch, W1, b1, W2, b2, Wfc, bfc):
    src = edge_index[0].astype(jnp.int32)
    dst = edge_index[1].astype(jnp.int32)
    padk = jnp.full((E_PAD - E,), N, dtype=jnp.int32)
    src_pad = jnp.concatenate([src, padk])
    dst_pad = jnp.concatenate([dst, padk])
    x_pad = jnp.pad(x, ((0, N_PAD - N), (0, 0)))
    batch_pad = jnp.concatenate(
        [batch.astype(jnp.int32), jnp.full((N_PAD - N,), G, dtype=jnp.int32)]
    ).reshape(_NB, 1, RA)
    zc = jnp.zeros((ROWS_PER_SUB, D), jnp.float32)
    onesb = jnp.ones((EB, D), jnp.float32)

    deg16 = _deg_sc(dst_pad, zc, onesb)
    u1, dinv = _lin1(x_pad, W1, deg16)
    agg1 = _agg_sc(u1, src_pad, dst_pad, zc)
    u2 = _layer2(agg1, u1, dinv, b1.reshape(1, D), W2)
    agg2 = _agg_sc(u2, src_pad, dst_pad, zc)
    out = _final(agg2, u2, dinv, b2.reshape(1, D), batch_pad,
                 Wfc, bfc.reshape(1, DOUT))
    return out


# speedup vs baseline: 9.0287x; 1.3047x over previous
import jax
import jax.numpy as jnp
from jax import lax
from jax.experimental import pallas as pl
from jax.experimental.pallas import tpu as pltpu
from jax.experimental.pallas import tpu_sc as plsc

N = 10000
E = 320000
D = 128
G = 128
DOUT = 16

NC = 2
NS = 16
NW = NC * NS
N_PAD = 10240
ROWS_PER_SUB = N_PAD // NS
EB = 128
E_PAD = 327680
EPT = E_PAD // NW
NBLK = EPT // EB
CHUNK = 8
ZR = 64


def _zero_acc_slice(zb_hbm, zb_v, acc, s):
    pltpu.sync_copy(zb_hbm, zb_v)

    @pl.loop(0, ROWS_PER_SUB // ZR)
    def _(j):
        pltpu.sync_copy(zb_v, acc.at[pl.ds(s * ROWS_PER_SUB + j * ZR, ZR)])


def _deg_body(dst_hbm, zb_hbm, ones_hbm, out_hbm, didx, ones_v, zb_v, acc):
    c = lax.axis_index("c")
    s = lax.axis_index("s")
    wid = c * NS + s
    pltpu.sync_copy(ones_hbm, ones_v)
    _zero_acc_slice(zb_hbm, zb_v, acc, s)
    plsc.subcore_barrier()

    @pl.loop(0, NBLK // CHUNK)
    def _(g):
        pltpu.sync_copy(dst_hbm.at[wid, pl.ds(g * CHUNK, CHUNK)], didx)

        @pl.loop(0, CHUNK)
        def _(b):
            pltpu.sync_copy(ones_v, acc.at[didx.at[b]], add=True)

    plsc.subcore_barrier()
    sl = pl.ds(s * ROWS_PER_SUB, ROWS_PER_SUB)
    pltpu.sync_copy(acc.at[sl], out_hbm.at[c, sl])


def _agg_body(u_hbm, src_hbm, dst_hbm, zb_hbm, out_hbm,
              sidx, didx, rows0, rows1, zb_v, sem0, sem1, acc):
    c = lax.axis_index("c")
    s = lax.axis_index("s")
    wid = c * NS + s
    _zero_acc_slice(zb_hbm, zb_v, acc, s)
    plsc.subcore_barrier()

    @pl.loop(0, NBLK // CHUNK)
    def _(g):
        pltpu.sync_copy(src_hbm.at[wid, pl.ds(g * CHUNK, CHUNK)], sidx)
        pltpu.sync_copy(dst_hbm.at[wid, pl.ds(g * CHUNK, CHUNK)], didx)

        pltpu.make_async_copy(u_hbm.at[sidx.at[0]], rows0, sem0).start()

        @pl.loop(0, CHUNK, step=2)
        def _(b):
            pltpu.make_async_copy(u_hbm.at[sidx.at[b + 1]], rows1, sem1).start()
            pltpu.make_async_copy(u_hbm.at[sidx.at[b]], rows0, sem0).wait()
            pltpu.sync_copy(rows0, acc.at[didx.at[b]], add=True)

            @pl.when(b + 2 < CHUNK)
            def _():
                pltpu.make_async_copy(
                    u_hbm.at[sidx.at[b + 2]], rows0, sem0).start()

            pltpu.make_async_copy(u_hbm.at[sidx.at[b + 1]], rows1, sem1).wait()
            pltpu.sync_copy(rows1, acc.at[didx.at[b + 1]], add=True)

    plsc.subcore_barrier()
    sl = pl.ds(s * ROWS_PER_SUB, ROWS_PER_SUB)
    pltpu.sync_copy(acc.at[sl], out_hbm.at[c, sl])


_SC_MESH = plsc.VectorSubcoreMesh(core_axis_name="c", subcore_axis_name="s")

_deg_sc = pl.kernel(
    _deg_body,
    out_type=jax.ShapeDtypeStruct((NC, N_PAD, D), jnp.float32),
    mesh=_SC_MESH,
    scratch_types=[
        pltpu.VMEM((CHUNK, EB), jnp.int32),
        pltpu.VMEM((EB, D), jnp.float32),
        pltpu.VMEM((ZR, D), jnp.float32),
        pltpu.VMEM_SHARED((N_PAD, D), jnp.float32),
    ],
)

_agg_sc = pl.kernel(
    _agg_body,
    out_type=jax.ShapeDtypeStruct((NC, N_PAD, D), jnp.float32),
    mesh=_SC_MESH,
    scratch_types=[
        pltpu.VMEM((CHUNK, EB), jnp.int32),
        pltpu.VMEM((CHUNK, EB), jnp.int32),
        pltpu.VMEM((EB, D), jnp.float32),
        pltpu.VMEM((EB, D), jnp.float32),
        pltpu.VMEM((ZR, D), jnp.float32),
        pltpu.SemaphoreType.DMA,
        pltpu.SemaphoreType.DMA,
        pltpu.VMEM_SHARED((N_PAD, D), jnp.float32),
    ],
)


RA = 2048


def _lin1_kern(x_ref, w_ref, deg_ref, u_ref, dinv_ref):
    deg = deg_ref[0, :, 0] + deg_ref[1, :, 0] + 1.0
    dinv = lax.rsqrt(deg)
    h = jnp.dot(x_ref[...], w_ref[...], preferred_element_type=jnp.float32)
    u_ref[...] = h * dinv[:, None]
    dinv_ref[...] = dinv[:, None]


def _layer2_kern(agg_ref, u_ref, dinv_ref, b1_ref, w_ref, u2_ref):
    a = agg_ref[0] + agg_ref[1] + u_ref[...]
    h = jnp.maximum(a * dinv_ref[...] + b1_ref[...], 0.0)
    u2_ref[...] = jnp.dot(h, w_ref[...],
                          preferred_element_type=jnp.float32) * dinv_ref[...]


def _final_kern(agg_ref, u_ref, dinv_ref, b2_ref, batch_ref, wfc_ref, bfc_ref,
                out_ref, pool_ref, cnt_ref):
    i = pl.program_id(0)

    @pl.when(i == 0)
    def _():
        pool_ref[...] = jnp.zeros_like(pool_ref)
        cnt_ref[...] = jnp.zeros_like(cnt_ref)

    a = agg_ref[0] + agg_ref[1] + u_ref[...]
    h = jnp.maximum(a * dinv_ref[...] + b2_ref[...], 0.0)
    b = batch_ref[0]
    rows = lax.broadcasted_iota(jnp.int32, (G, RA), 0)
    oh = (rows == b).astype(jnp.float32)
    pool_ref[...] += jnp.dot(oh, h, preferred_element_type=jnp.float32)
    cnt_ref[...] += jnp.dot(oh, jnp.ones((RA, 1), jnp.float32),
                            preferred_element_type=jnp.float32)

    @pl.when(i == pl.num_programs(0) - 1)
    def _():
        g = pool_ref[...] / jnp.maximum(cnt_ref[...], 1.0)
        logits = jnp.dot(g, wfc_ref[...],
                         preferred_element_type=jnp.float32) + bfc_ref[...]
        m = jnp.max(logits, axis=1, keepdims=True)
        e = jnp.exp(logits - m)
        out_ref[...] = (logits - m) - jnp.log(jnp.sum(e, axis=1, keepdims=True))


_NB = N_PAD // RA

_lin1 = pl.pallas_call(
    _lin1_kern,
    grid=(_NB,),
    in_specs=[
        pl.BlockSpec((RA, D), lambda i: (i, 0)),
        pl.BlockSpec((D, D), lambda i: (0, 0)),
        pl.BlockSpec((NC, RA, D), lambda i: (0, i, 0)),
    ],
    out_specs=[
        pl.BlockSpec((RA, D), lambda i: (i, 0)),
        pl.BlockSpec((RA, 1), lambda i: (i, 0)),
    ],
    out_shape=[
        jax.ShapeDtypeStruct((N_PAD, D), jnp.float32),
        jax.ShapeDtypeStruct((N_PAD, 1), jnp.float32),
    ],
)

_layer2 = pl.pallas_call(
    _layer2_kern,
    grid=(_NB,),
    in_specs=[
        pl.BlockSpec((NC, RA, D), lambda i: (0, i, 0)),
        pl.BlockSpec((RA, D), lambda i: (i, 0)),
        pl.BlockSpec((RA, 1), lambda i: (i, 0)),
        pl.BlockSpec((1, D), lambda i: (0, 0)),
        pl.BlockSpec((D, D), lambda i: (0, 0)),
    ],
    out_specs=pl.BlockSpec((RA, D), lambda i: (i, 0)),
    out_shape=jax.ShapeDtypeStruct((N_PAD, D), jnp.float32),
)

_final = pl.pallas_call(
    _final_kern,
    grid=(_NB,),
    in_specs=[
        pl.BlockSpec((NC, RA, D), lambda i: (0, i, 0)),
        pl.BlockSpec((RA, D), lambda i: (i, 0)),
        pl.BlockSpec((RA, 1), lambda i: (i, 0)),
        pl.BlockSpec((1, D), lambda i: (0, 0)),
        pl.BlockSpec((1, 1, RA), lambda i: (i, 0, 0)),
        pl.BlockSpec((D, DOUT), lambda i: (0, 0)),
        pl.BlockSpec((1, DOUT), lambda i: (0, 0)),
    ],
    out_specs=pl.BlockSpec((G, DOUT), lambda i: (0, 0)),
    out_shape=jax.ShapeDtypeStruct((G, DOUT), jnp.float32),
    scratch_shapes=[
        pltpu.VMEM((G, G), jnp.float32),
        pltpu.VMEM((G, 1), jnp.float32),
    ],
)


def kernel(x, edge_index, batch, W1, b1, W2, b2, Wfc, bfc):
    src = edge_index[0].astype(jnp.int32)
    dst = edge_index[1].astype(jnp.int32)
    padk = jnp.full((E_PAD - E,), N, dtype=jnp.int32)
    src3 = jnp.concatenate([src, padk]).reshape(NW, NBLK, EB)
    dst3 = jnp.concatenate([dst, padk]).reshape(NW, NBLK, EB)
    x_pad = jnp.pad(x, ((0, N_PAD - N), (0, 0)))
    batch_pad = jnp.concatenate(
        [batch.astype(jnp.int32), jnp.full((N_PAD - N,), G, dtype=jnp.int32)]
    ).reshape(_NB, 1, RA)
    zb = jnp.zeros((ZR, D), jnp.float32)
    onesb = jnp.ones((EB, D), jnp.float32)

    deg16 = _deg_sc(dst3, zb, onesb)
    u1, dinv = _lin1(x_pad, W1, deg16)
    agg1 = _agg_sc(u1, src3, dst3, zb)
    u2 = _layer2(agg1, u1, dinv, b1.reshape(1, D), W2)
    agg2 = _agg_sc(u2, src3, dst3, zb)
    out = _final(agg2, u2, dinv, b2.reshape(1, D), batch_pad,
                 Wfc, bfc.reshape(1, DOUT))
    return out


# speedup vs baseline: 24.8902x; 2.7568x over previous
import jax
import jax.numpy as jnp
from jax import lax
from jax.experimental import pallas as pl
from jax.experimental.pallas import tpu as pltpu
from jax.experimental.pallas import tpu_sc as plsc

N = 10000
E = 320000
D = 128
G = 128
DOUT = 16

NC = 2
NS = 16
NW = NC * NS
N_PAD = 10240
ROWS_PER_SUB = N_PAD // NS
EB = 128
E_PAD = 327680
EPT = E_PAD // NW
NBLK = EPT // EB
CHUNK = 8
ZR = 64


def _zero_acc_slice(zb_hbm, zb_v, acc, s):
    pltpu.sync_copy(zb_hbm, zb_v)

    @pl.loop(0, ROWS_PER_SUB // ZR)
    def _(j):
        pltpu.sync_copy(zb_v, acc.at[pl.ds(s * ROWS_PER_SUB + j * ZR, ZR)])


def _deg_body(dst_hbm, zb_hbm, ones_hbm, out_hbm, didx, ones_v, zb_v, acc):
    c = lax.axis_index("c")
    s = lax.axis_index("s")
    wid = c * NS + s
    pltpu.sync_copy(ones_hbm, ones_v)
    _zero_acc_slice(zb_hbm, zb_v, acc, s)
    plsc.subcore_barrier()

    @pl.loop(0, NBLK // CHUNK)
    def _(g):
        pltpu.sync_copy(dst_hbm.at[wid, pl.ds(g * CHUNK, CHUNK)], didx)

        @pl.loop(0, CHUNK)
        def _(b):
            pltpu.sync_copy(ones_v, acc.at[didx.at[b]], add=True)

    plsc.subcore_barrier()
    sl = pl.ds(s * ROWS_PER_SUB, ROWS_PER_SUB)
    pltpu.sync_copy(acc.at[sl], out_hbm.at[c, sl])


def _agg_body(u_hbm, src_hbm, dst_hbm, zb_hbm, out_hbm,
              sidx, didx, rows0, rows1, zb_v, sem0, sem1, acc):
    c = lax.axis_index("c")
    s = lax.axis_index("s")
    wid = c * NS + s
    _zero_acc_slice(zb_hbm, zb_v, acc, s)
    plsc.subcore_barrier()

    @pl.loop(0, NBLK // CHUNK)
    def _(g):
        pltpu.sync_copy(src_hbm.at[wid, pl.ds(g * CHUNK, CHUNK)], sidx)
        pltpu.sync_copy(dst_hbm.at[wid, pl.ds(g * CHUNK, CHUNK)], didx)

        pltpu.make_async_copy(u_hbm.at[sidx.at[0]], rows0, sem0).start()

        @pl.loop(0, CHUNK, step=2)
        def _(b):
            pltpu.make_async_copy(u_hbm.at[sidx.at[b + 1]], rows1, sem1).start()
            pltpu.make_async_copy(u_hbm.at[sidx.at[b]], rows0, sem0).wait()
            pltpu.sync_copy(rows0, acc.at[didx.at[b]], add=True)

            @pl.when(b + 2 < CHUNK)
            def _():
                pltpu.make_async_copy(
                    u_hbm.at[sidx.at[b + 2]], rows0, sem0).start()

            pltpu.make_async_copy(u_hbm.at[sidx.at[b + 1]], rows1, sem1).wait()
            pltpu.sync_copy(rows1, acc.at[didx.at[b + 1]], add=True)

    plsc.subcore_barrier()
    sl = pl.ds(s * ROWS_PER_SUB, ROWS_PER_SUB)
    pltpu.sync_copy(acc.at[sl], out_hbm.at[c, sl])


_SC_MESH = plsc.VectorSubcoreMesh(core_axis_name="c", subcore_axis_name="s")

_deg_sc = pl.kernel(
    _deg_body,
    out_type=jax.ShapeDtypeStruct((NC, N_PAD, D), jnp.float32),
    mesh=_SC_MESH,
    scratch_types=[
        pltpu.VMEM((CHUNK, EB), jnp.int32),
        pltpu.VMEM((EB, D), jnp.float32),
        pltpu.VMEM((ZR, D), jnp.float32),
        pltpu.VMEM_SHARED((N_PAD, D), jnp.float32),
    ],
)

_agg_sc = pl.kernel(
    _agg_body,
    out_type=jax.ShapeDtypeStruct((NC, N_PAD, D), jnp.float32),
    mesh=_SC_MESH,
    scratch_types=[
        pltpu.VMEM((CHUNK, EB), jnp.int32),
        pltpu.VMEM((CHUNK, EB), jnp.int32),
        pltpu.VMEM((EB, D), jnp.float32),
        pltpu.VMEM((EB, D), jnp.float32),
        pltpu.VMEM((ZR, D), jnp.float32),
        pltpu.SemaphoreType.DMA,
        pltpu.SemaphoreType.DMA,
        pltpu.VMEM_SHARED((N_PAD, D), jnp.float32),
    ],
)


RA = 2048


def _lin1_kern(x_ref, w_ref, deg_ref, u_ref, dinv_ref):
    deg = deg_ref[0, :, 0] + deg_ref[1, :, 0] + 1.0
    dinv = lax.rsqrt(deg)
    h = jnp.dot(x_ref[...], w_ref[...], preferred_element_type=jnp.float32)
    u_ref[...] = h * dinv[:, None]
    dinv_ref[...] = dinv[:, None]


def _layer2_kern(agg_ref, u_ref, dinv_ref, b1_ref, w_ref, u2_ref):
    a = agg_ref[0] + agg_ref[1] + u_ref[...]
    h = jnp.maximum(a * dinv_ref[...] + b1_ref[...], 0.0)
    u2_ref[...] = jnp.dot(h, w_ref[...],
                          preferred_element_type=jnp.float32) * dinv_ref[...]


def _final_kern(agg_ref, u_ref, dinv_ref, b2_ref, batch_ref, wfc_ref, bfc_ref,
                out_ref, pool_ref, cnt_ref):
    i = pl.program_id(0)

    @pl.when(i == 0)
    def _():
        pool_ref[...] = jnp.zeros_like(pool_ref)
        cnt_ref[...] = jnp.zeros_like(cnt_ref)

    a = agg_ref[0] + agg_ref[1] + u_ref[...]
    h = jnp.maximum(a * dinv_ref[...] + b2_ref[...], 0.0)
    b = batch_ref[0]
    rows = lax.broadcasted_iota(jnp.int32, (G, RA), 0)
    oh = (rows == b).astype(jnp.float32)
    pool_ref[...] += jnp.dot(oh, h, preferred_element_type=jnp.float32)
    cnt_ref[...] += jnp.dot(oh, jnp.ones((RA, 1), jnp.float32),
                            preferred_element_type=jnp.float32)

    @pl.when(i == pl.num_programs(0) - 1)
    def _():
        g = pool_ref[...] / jnp.maximum(cnt_ref[...], 1.0)
        logits = jnp.dot(g, wfc_ref[...],
                         preferred_element_type=jnp.float32) + bfc_ref[...]
        m = jnp.max(logits, axis=1, keepdims=True)
        e = jnp.exp(logits - m)
        out_ref[...] = (logits - m) - jnp.log(jnp.sum(e, axis=1, keepdims=True))


_NB = N_PAD // RA

_lin1 = pl.pallas_call(
    _lin1_kern,
    grid=(_NB,),
    in_specs=[
        pl.BlockSpec((RA, D), lambda i: (i, 0)),
        pl.BlockSpec((D, D), lambda i: (0, 0)),
        pl.BlockSpec((NC, RA, D), lambda i: (0, i, 0)),
    ],
    out_specs=[
        pl.BlockSpec((RA, D), lambda i: (i, 0)),
        pl.BlockSpec((RA, 1), lambda i: (i, 0)),
    ],
    out_shape=[
        jax.ShapeDtypeStruct((N_PAD, D), jnp.float32),
        jax.ShapeDtypeStruct((N_PAD, 1), jnp.float32),
    ],
)

_layer2 = pl.pallas_call(
    _layer2_kern,
    grid=(_NB,),
    in_specs=[
        pl.BlockSpec((NC, RA, D), lambda i: (0, i, 0)),
        pl.BlockSpec((RA, D), lambda i: (i, 0)),
        pl.BlockSpec((RA, 1), lambda i: (i, 0)),
        pl.BlockSpec((1, D), lambda i: (0, 0)),
        pl.BlockSpec((D, D), lambda i: (0, 0)),
    ],
    out_specs=pl.BlockSpec((RA, D), lambda i: (i, 0)),
    out_shape=jax.ShapeDtypeStruct((N_PAD, D), jnp.float32),
)

_final = pl.pallas_call(
    _final_kern,
    grid=(_NB,),
    in_specs=[
        pl.BlockSpec((NC, RA, D), lambda i: (0, i, 0)),
        pl.BlockSpec((RA, D), lambda i: (i, 0)),
        pl.BlockSpec((RA, 1), lambda i: (i, 0)),
        pl.BlockSpec((1, D), lambda i: (0, 0)),
        pl.BlockSpec((1, 1, RA), lambda i: (i, 0, 0)),
        pl.BlockSpec((D, DOUT), lambda i: (0, 0)),
        pl.BlockSpec((1, DOUT), lambda i: (0, 0)),
    ],
    out_specs=pl.BlockSpec((G, DOUT), lambda i: (0, 0)),
    out_shape=jax.ShapeDtypeStruct((G, DOUT), jnp.float32),
    scratch_shapes=[
        pltpu.VMEM((G, G), jnp.float32),
        pltpu.VMEM((G, 1), jnp.float32),
    ],
)


def kernel(x, edge_index, batch, W1, b1, W2, b2, Wfc, bfc):
    src = edge_index[0].astype(jnp.int32)
    dst = edge_index[1].astype(jnp.int32)
    padk = N + jnp.arange(E_PAD - E, dtype=jnp.int32) % (N_PAD - N)
    src3 = jnp.concatenate([src, padk]).reshape(NW, NBLK, EB)
    dst3 = jnp.concatenate([dst, padk]).reshape(NW, NBLK, EB)
    x_pad = jnp.pad(x, ((0, N_PAD - N), (0, 0)))
    batch_pad = jnp.concatenate(
        [batch.astype(jnp.int32), jnp.full((N_PAD - N,), G, dtype=jnp.int32)]
    ).reshape(_NB, 1, RA)
    zb = jnp.zeros((ZR, D), jnp.float32)
    onesb = jnp.ones((EB, D), jnp.float32)

    deg16 = _deg_sc(dst3, zb, onesb)
    u1, dinv = _lin1(x_pad, W1, deg16)
    agg1 = _agg_sc(u1, src3, dst3, zb)
    u2 = _layer2(agg1, u1, dinv, b1.reshape(1, D), W2)
    agg2 = _agg_sc(u2, src3, dst3, zb)
    out = _final(agg2, u2, dinv, b2.reshape(1, D), batch_pad,
                 Wfc, bfc.reshape(1, DOUT))
    return out


# speedup vs baseline: 26.6744x; 1.0717x over previous
import jax
import jax.numpy as jnp
from jax import lax
from jax.experimental import pallas as pl
from jax.experimental.pallas import tpu as pltpu
from jax.experimental.pallas import tpu_sc as plsc

N = 10000
E = 320000
D = 128
G = 128
DOUT = 16

NC = 2
NS = 16
NW = NC * NS
N_PAD = 10240
ROWS_PER_SUB = N_PAD // NS
EB = 128
E_PAD = 327680
EPT = E_PAD // NW
NBLK = EPT // EB
CHUNK = 8
NCH = NBLK // CHUNK
ZR = 64


def _zero_acc_slice(zb_hbm, zb_v, acc, s):
    pltpu.sync_copy(zb_hbm, zb_v)

    @pl.loop(0, ROWS_PER_SUB // ZR)
    def _(j):
        pltpu.sync_copy(zb_v, acc.at[pl.ds(s * ROWS_PER_SUB + j * ZR, ZR)])


def _deg_body(dst_hbm, zb_hbm, ones_hbm, out_hbm, didx0, didx1, ones_v, zb_v,
              isem0, isem1, acc):
    c = lax.axis_index("c")
    s = lax.axis_index("s")
    wid = c * NS + s
    pltpu.sync_copy(ones_hbm, ones_v)
    _zero_acc_slice(zb_hbm, zb_v, acc, s)
    pltpu.make_async_copy(dst_hbm.at[wid, pl.ds(0, CHUNK)], didx0, isem0).start()
    plsc.subcore_barrier()

    @pl.loop(0, NCH, step=2)
    def _(g):
        pltpu.make_async_copy(
            dst_hbm.at[wid, pl.ds((g + 1) * CHUNK, CHUNK)], didx1, isem1).start()
        pltpu.make_async_copy(
            dst_hbm.at[wid, pl.ds(g * CHUNK, CHUNK)], didx0, isem0).wait()

        @pl.loop(0, CHUNK)
        def _(b):
            pltpu.sync_copy(ones_v, acc.at[didx0.at[b]], add=True)

        @pl.when(g + 2 < NCH)
        def _():
            pltpu.make_async_copy(
                dst_hbm.at[wid, pl.ds((g + 2) * CHUNK, CHUNK)], didx0,
                isem0).start()

        pltpu.make_async_copy(
            dst_hbm.at[wid, pl.ds((g + 1) * CHUNK, CHUNK)], didx1, isem1).wait()

        @pl.loop(0, CHUNK)
        def _(b):
            pltpu.sync_copy(ones_v, acc.at[didx1.at[b]], add=True)

    plsc.subcore_barrier()
    sl = pl.ds(s * ROWS_PER_SUB, ROWS_PER_SUB)
    pltpu.sync_copy(acc.at[sl], out_hbm.at[c, sl])


def _agg_chunk(u_hbm, sidx, didx, rows0, rows1, sem0, sem1, acc):
    pltpu.make_async_copy(u_hbm.at[sidx.at[0]], rows0, sem0).start()

    @pl.loop(0, CHUNK, step=2)
    def _(b):
        pltpu.make_async_copy(u_hbm.at[sidx.at[b + 1]], rows1, sem1).start()
        pltpu.make_async_copy(u_hbm.at[sidx.at[b]], rows0, sem0).wait()
        pltpu.sync_copy(rows0, acc.at[didx.at[b]], add=True)

        @pl.when(b + 2 < CHUNK)
        def _():
            pltpu.make_async_copy(
                u_hbm.at[sidx.at[b + 2]], rows0, sem0).start()

        pltpu.make_async_copy(u_hbm.at[sidx.at[b + 1]], rows1, sem1).wait()
        pltpu.sync_copy(rows1, acc.at[didx.at[b + 1]], add=True)


def _agg_body(u_hbm, src_hbm, dst_hbm, zb_hbm, out_hbm,
              sidx0, didx0, sidx1, didx1, rows0, rows1, zb_v,
              isem0, isem1, sem0, sem1, acc):
    c = lax.axis_index("c")
    s = lax.axis_index("s")
    wid = c * NS + s
    _zero_acc_slice(zb_hbm, zb_v, acc, s)
    pltpu.make_async_copy(src_hbm.at[wid, pl.ds(0, CHUNK)], sidx0, isem0).start()
    pltpu.make_async_copy(dst_hbm.at[wid, pl.ds(0, CHUNK)], didx0, isem0).start()
    plsc.subcore_barrier()

    @pl.loop(0, NCH, step=2)
    def _(g):
        pltpu.make_async_copy(
            src_hbm.at[wid, pl.ds((g + 1) * CHUNK, CHUNK)], sidx1, isem1).start()
        pltpu.make_async_copy(
            dst_hbm.at[wid, pl.ds((g + 1) * CHUNK, CHUNK)], didx1, isem1).start()
        pltpu.make_async_copy(
            src_hbm.at[wid, pl.ds(g * CHUNK, CHUNK)], sidx0, isem0).wait()
        pltpu.make_async_copy(
            dst_hbm.at[wid, pl.ds(g * CHUNK, CHUNK)], didx0, isem0).wait()
        _agg_chunk(u_hbm, sidx0, didx0, rows0, rows1, sem0, sem1, acc)

        @pl.when(g + 2 < NCH)
        def _():
            pltpu.make_async_copy(
                src_hbm.at[wid, pl.ds((g + 2) * CHUNK, CHUNK)], sidx0,
                isem0).start()
            pltpu.make_async_copy(
                dst_hbm.at[wid, pl.ds((g + 2) * CHUNK, CHUNK)], didx0,
                isem0).start()

        pltpu.make_async_copy(
            src_hbm.at[wid, pl.ds((g + 1) * CHUNK, CHUNK)], sidx1, isem1).wait()
        pltpu.make_async_copy(
            dst_hbm.at[wid, pl.ds((g + 1) * CHUNK, CHUNK)], didx1, isem1).wait()
        _agg_chunk(u_hbm, sidx1, didx1, rows0, rows1, sem0, sem1, acc)

    plsc.subcore_barrier()
    sl = pl.ds(s * ROWS_PER_SUB, ROWS_PER_SUB)
    pltpu.sync_copy(acc.at[sl], out_hbm.at[c, sl])


_SC_MESH = plsc.VectorSubcoreMesh(core_axis_name="c", subcore_axis_name="s")

_deg_sc = pl.kernel(
    _deg_body,
    out_type=jax.ShapeDtypeStruct((NC, N_PAD, D), jnp.float32),
    mesh=_SC_MESH,
    scratch_types=[
        pltpu.VMEM((CHUNK, EB), jnp.int32),
        pltpu.VMEM((CHUNK, EB), jnp.int32),
        pltpu.VMEM((EB, D), jnp.float32),
        pltpu.VMEM((ZR, D), jnp.float32),
        pltpu.SemaphoreType.DMA,
        pltpu.SemaphoreType.DMA,
        pltpu.VMEM_SHARED((N_PAD, D), jnp.float32),
    ],
)

_agg_sc = pl.kernel(
    _agg_body,
    out_type=jax.ShapeDtypeStruct((NC, N_PAD, D), jnp.float32),
    mesh=_SC_MESH,
    scratch_types=[
        pltpu.VMEM((CHUNK, EB), jnp.int32),
        pltpu.VMEM((CHUNK, EB), jnp.int32),
        pltpu.VMEM((CHUNK, EB), jnp.int32),
        pltpu.VMEM((CHUNK, EB), jnp.int32),
        pltpu.VMEM((EB, D), jnp.float32),
        pltpu.VMEM((EB, D), jnp.float32),
        pltpu.VMEM((ZR, D), jnp.float32),
        pltpu.SemaphoreType.DMA,
        pltpu.SemaphoreType.DMA,
        pltpu.SemaphoreType.DMA,
        pltpu.SemaphoreType.DMA,
        pltpu.VMEM_SHARED((N_PAD, D), jnp.float32),
    ],
)


RA = 2048


def _lin1_kern(x_ref, w_ref, deg_ref, u_ref, dinv_ref):
    deg = deg_ref[0, :, 0] + deg_ref[1, :, 0] + 1.0
    dinv = lax.rsqrt(deg)
    h = jnp.dot(x_ref[...], w_ref[...], preferred_element_type=jnp.float32)
    u_ref[...] = h * dinv[:, None]
    dinv_ref[...] = dinv[:, None]


def _layer2_kern(agg_ref, u_ref, dinv_ref, b1_ref, w_ref, u2_ref):
    a = agg_ref[0] + agg_ref[1] + u_ref[...]
    h = jnp.maximum(a * dinv_ref[...] + b1_ref[...], 0.0)
    u2_ref[...] = jnp.dot(h, w_ref[...],
                          preferred_element_type=jnp.float32) * dinv_ref[...]


def _final_kern(agg_ref, u_ref, dinv_ref, b2_ref, batch_ref, wfc_ref, bfc_ref,
                out_ref, pool_ref, cnt_ref):
    i = pl.program_id(0)

    @pl.when(i == 0)
    def _():
        pool_ref[...] = jnp.zeros_like(pool_ref)
        cnt_ref[...] = jnp.zeros_like(cnt_ref)

    a = agg_ref[0] + agg_ref[1] + u_ref[...]
    h = jnp.maximum(a * dinv_ref[...] + b2_ref[...], 0.0)
    b = batch_ref[0]
    rows = lax.broadcasted_iota(jnp.int32, (G, RA), 0)
    oh = (rows == b).astype(jnp.float32)
    pool_ref[...] += jnp.dot(oh, h, preferred_element_type=jnp.float32)
    cnt_ref[...] += jnp.dot(oh, jnp.ones((RA, 1), jnp.float32),
                            preferred_element_type=jnp.float32)

    @pl.when(i == pl.num_programs(0) - 1)
    def _():
        g = pool_ref[...] / jnp.maximum(cnt_ref[...], 1.0)
        logits = jnp.dot(g, wfc_ref[...],
                         preferred_element_type=jnp.float32) + bfc_ref[...]
        m = jnp.max(logits, axis=1, keepdims=True)
        e = jnp.exp(logits - m)
        out_ref[...] = (logits - m) - jnp.log(jnp.sum(e, axis=1, keepdims=True))


_NB = N_PAD // RA

_lin1 = pl.pallas_call(
    _lin1_kern,
    grid=(_NB,),
    in_specs=[
        pl.BlockSpec((RA, D), lambda i: (i, 0)),
        pl.BlockSpec((D, D), lambda i: (0, 0)),
        pl.BlockSpec((NC, RA, D), lambda i: (0, i, 0)),
    ],
    out_specs=[
        pl.BlockSpec((RA, D), lambda i: (i, 0)),
        pl.BlockSpec((RA, 1), lambda i: (i, 0)),
    ],
    out_shape=[
        jax.ShapeDtypeStruct((N_PAD, D), jnp.float32),
        jax.ShapeDtypeStruct((N_PAD, 1), jnp.float32),
    ],
)

_layer2 = pl.pallas_call(
    _layer2_kern,
    grid=(_NB,),
    in_specs=[
        pl.BlockSpec((NC, RA, D), lambda i: (0, i, 0)),
        pl.BlockSpec((RA, D), lambda i: (i, 0)),
        pl.BlockSpec((RA, 1), lambda i: (i, 0)),
        pl.BlockSpec((1, D), lambda i: (0, 0)),
        pl.BlockSpec((D, D), lambda i: (0, 0)),
    ],
    out_specs=pl.BlockSpec((RA, D), lambda i: (i, 0)),
    out_shape=jax.ShapeDtypeStruct((N_PAD, D), jnp.float32),
)

_final = pl.pallas_call(
    _final_kern,
    grid=(_NB,),
    in_specs=[
        pl.BlockSpec((NC, RA, D), lambda i: (0, i, 0)),
        pl.BlockSpec((RA, D), lambda i: (i, 0)),
        pl.BlockSpec((RA, 1), lambda i: (i, 0)),
        pl.BlockSpec((1, D), lambda i: (0, 0)),
        pl.BlockSpec((1, 1, RA), lambda i: (i, 0, 0)),
        pl.BlockSpec((D, DOUT), lambda i: (0, 0)),
        pl.BlockSpec((1, DOUT), lambda i: (0, 0)),
    ],
    out_specs=pl.BlockSpec((G, DOUT), lambda i: (0, 0)),
    out_shape=jax.ShapeDtypeStruct((G, DOUT), jnp.float32),
    scratch_shapes=[
        pltpu.VMEM((G, G), jnp.float32),
        pltpu.VMEM((G, 1), jnp.float32),
    ],
)


def kernel(x, edge_index, batch, W1, b1, W2, b2, Wfc, bfc):
    src = edge_index[0].astype(jnp.int32)
    dst = edge_index[1].astype(jnp.int32)
    padk = N + jnp.arange(E_PAD - E, dtype=jnp.int32) % (N_PAD - N)
    src3 = jnp.concatenate([src, padk]).reshape(NW, NBLK, EB)
    dst3 = jnp.concatenate([dst, padk]).reshape(NW, NBLK, EB)
    x_pad = jnp.pad(x, ((0, N_PAD - N), (0, 0)))
    batch_pad = jnp.concatenate(
        [batch.astype(jnp.int32), jnp.full((N_PAD - N,), G, dtype=jnp.int32)]
    ).reshape(_NB, 1, RA)
    zb = jnp.zeros((ZR, D), jnp.float32)
    onesb = jnp.ones((EB, D), jnp.float32)

    deg16 = _deg_sc(dst3, zb, onesb)
    u1, dinv = _lin1(x_pad, W1, deg16)
    agg1 = _agg_sc(u1, src3, dst3, zb)
    u2 = _layer2(agg1, u1, dinv, b1.reshape(1, D), W2)
    agg2 = _agg_sc(u2, src3, dst3, zb)
    out = _final(agg2, u2, dinv, b2.reshape(1, D), batch_pad,
                 Wfc, bfc.reshape(1, DOUT))
    return out
